# baseline (speedup 1.0000x reference)
import jax
import jax.numpy as jnp
from jax.experimental import pallas as pl
from jax.experimental.pallas import tpu as pltpu

NNDR_R = 0.8
KBLK = 4096
STATE_W = 128
PAD_VAL = 1000.0


def _pair_merge(ll, hl, lr, hr):
    lo = jnp.minimum(ll, lr)
    hi = jnp.minimum(jnp.maximum(ll, lr), jnp.minimum(hl, hr))
    return lo, hi


def _nndr_kernel(qs_ref, ka_ref, out_ref, lo_ref, hi_ref):
    i = pl.program_id(0)
    nblk = pl.num_programs(0)

    @pl.when(i == 0)
    def _init():
        lo_ref[...] = jnp.full(lo_ref.shape, jnp.inf, lo_ref.dtype)
        hi_ref[...] = jnp.full(hi_ref.shape, jnp.inf, hi_ref.dtype)

    qs = qs_ref[:, :-1]
    qn = qs_ref[:, -1:]
    dot2 = jnp.dot(qs, ka_ref[:-1, :], preferred_element_type=jnp.float32)
    t = (qn + ka_ref[-1:, :]) + dot2

    w = t.shape[1] // 2
    lo = jnp.minimum(t[:, :w], t[:, w:])
    hi = jnp.maximum(t[:, :w], t[:, w:])
    while w > STATE_W:
        w //= 2
        lo, hi = _pair_merge(lo[:, :w], hi[:, :w], lo[:, w:], hi[:, w:])

    lo, hi = _pair_merge(lo_ref[...], hi_ref[...], lo, hi)
    lo_ref[...] = lo
    hi_ref[...] = hi

    @pl.when(i == nblk - 1)
    def _final():
        flo, fhi = lo, hi
        fw = STATE_W
        while fw > 8:
            fw //= 2
            flo, fhi = _pair_merge(flo[:, :fw], fhi[:, :fw],
                                   flo[:, fw:], fhi[:, fw:])
        bm1 = jnp.min(flo, axis=1, keepdims=True)
        eq = flo == bm1
        nmin = jnp.sum(eq.astype(jnp.float32), axis=1, keepdims=True)
        lo2 = jnp.min(jnp.where(eq, jnp.inf, flo), axis=1, keepdims=True)
        hi1 = jnp.min(jnp.where(eq, fhi, jnp.inf), axis=1, keepdims=True)
        bm2 = jnp.where(nmin > 1.0, bm1, jnp.minimum(lo2, hi1))

        s1 = jnp.maximum(bm1, 0.0)
        s2 = jnp.maximum(bm2, 0.0)
        d1 = jnp.sqrt(s1)
        d2 = jnp.sqrt(s2)
        mask = d1 < NNDR_R * d2
        per = jnp.sqrt(s1 + 1e-12)
        cnt = jnp.sum(mask.astype(jnp.float32))
        tot = jnp.sum(jnp.where(mask, per, 0.0))
        loss = jnp.where(cnt > 0.0, tot / jnp.maximum(cnt, 1.0), 0.0)
        out_ref[...] = loss.reshape(1, 1)


def kernel(desc_nir, desc_rgb):
    q_n, d = desc_nir.shape
    k_n = desc_rgb.shape[0]
    nblk = pl.cdiv(k_n, KBLK)
    kp = nblk * KBLK
    if kp != k_n:
        pad = jnp.full((kp - k_n, d), PAD_VAL, desc_rgb.dtype)
        desc_rgb = jnp.concatenate([desc_rgb, pad], axis=0)
    kn = jnp.sum(desc_rgb * desc_rgb, axis=1)[None, :]
    ka = jnp.concatenate([desc_rgb.T, kn], axis=0)
    qn = jnp.sum(desc_nir * desc_nir, axis=1, keepdims=True)
    qa = jnp.concatenate([-2.0 * desc_nir, qn], axis=1)
    out = pl.pallas_call(
        _nndr_kernel,
        grid=(nblk,),
        in_specs=[pl.BlockSpec((q_n, d + 1), lambda i: (0, 0)),
                  pl.BlockSpec((d + 1, KBLK), lambda i: (0, i))],
        out_specs=pl.BlockSpec((1, 1), lambda i: (0, 0)),
        out_shape=jax.ShapeDtypeStruct((1, 1), jnp.float32),
        scratch_shapes=[pltpu.VMEM((q_n, STATE_W), jnp.float32),
                        pltpu.VMEM((q_n, STATE_W), jnp.float32)],
    )(qa, ka)
    return out[0, 0]


# speedup vs baseline: 1.1240x; 1.1240x over previous
import jax
import jax.numpy as jnp
from jax.experimental import pallas as pl
from jax.experimental.pallas import tpu as pltpu

NNDR_R = 0.8
KBLK = 4096
STATE_W = 128
PAD_VAL = 1000.0


def _pair_merge(ll, hl, lr, hr):
    lo = jnp.minimum(ll, lr)
    hi = jnp.minimum(jnp.maximum(ll, lr), jnp.minimum(hl, hr))
    return lo, hi


def _nndr_kernel(qs_ref, ka_ref, out_ref, lo_ref, hi_ref):
    i = pl.program_id(0)
    nblk = pl.num_programs(0)

    @pl.when(i == 0)
    def _init():
        lo_ref[...] = jnp.full(lo_ref.shape, jnp.inf, lo_ref.dtype)
        hi_ref[...] = jnp.full(hi_ref.shape, jnp.inf, hi_ref.dtype)

    qs = qs_ref[...]
    dot2 = jnp.dot(qs, ka_ref[:-1, :], preferred_element_type=jnp.float32)
    t = ka_ref[-1:, :] + dot2

    w = t.shape[1] // 2
    lo = jnp.minimum(t[:, :w], t[:, w:])
    hi = jnp.maximum(t[:, :w], t[:, w:])
    while w > STATE_W:
        w //= 2
        lo, hi = _pair_merge(lo[:, :w], hi[:, :w], lo[:, w:], hi[:, w:])

    lo, hi = _pair_merge(lo_ref[...], hi_ref[...], lo, hi)
    lo_ref[...] = lo
    hi_ref[...] = hi

    @pl.when(i == nblk - 1)
    def _final():
        flo, fhi = lo, hi
        fw = STATE_W
        while fw > 8:
            fw //= 2
            flo, fhi = _pair_merge(flo[:, :fw], fhi[:, :fw],
                                   flo[:, fw:], fhi[:, fw:])
        bm1 = jnp.min(flo, axis=1, keepdims=True)
        eq = flo == bm1
        nmin = jnp.sum(eq.astype(jnp.float32), axis=1, keepdims=True)
        lo2 = jnp.min(jnp.where(eq, jnp.inf, flo), axis=1, keepdims=True)
        hi1 = jnp.min(jnp.where(eq, fhi, jnp.inf), axis=1, keepdims=True)
        bm2 = jnp.where(nmin > 1.0, bm1, jnp.minimum(lo2, hi1))

        qn = 0.25 * jnp.sum(qs * qs, axis=1, keepdims=True)
        s1 = jnp.maximum(bm1 + qn, 0.0)
        s2 = jnp.maximum(bm2 + qn, 0.0)
        d1 = jnp.sqrt(s1)
        d2 = jnp.sqrt(s2)
        mask = d1 < NNDR_R * d2
        per = jnp.sqrt(s1 + 1e-12)
        cnt = jnp.sum(mask.astype(jnp.float32))
        tot = jnp.sum(jnp.where(mask, per, 0.0))
        loss = jnp.where(cnt > 0.0, tot / jnp.maximum(cnt, 1.0), 0.0)
        out_ref[...] = loss.reshape(1, 1)


def kernel(desc_nir, desc_rgb):
    q_n, d = desc_nir.shape
    k_n = desc_rgb.shape[0]
    nblk = pl.cdiv(k_n, KBLK)
    kp = nblk * KBLK
    if kp != k_n:
        pad = jnp.full((kp - k_n, d), PAD_VAL, desc_rgb.dtype)
        desc_rgb = jnp.concatenate([desc_rgb, pad], axis=0)
    kn = jnp.sum(desc_rgb * desc_rgb, axis=1)[None, :]
    ka = jnp.concatenate([desc_rgb.T, kn], axis=0)
    out = pl.pallas_call(
        _nndr_kernel,
        grid=(nblk,),
        in_specs=[pl.BlockSpec((q_n, d), lambda i: (0, 0)),
                  pl.BlockSpec((d + 1, KBLK), lambda i: (0, i))],
        out_specs=pl.BlockSpec((1, 1), lambda i: (0, 0)),
        out_shape=jax.ShapeDtypeStruct((1, 1), jnp.float32),
        scratch_shapes=[pltpu.VMEM((q_n, STATE_W), jnp.float32),
                        pltpu.VMEM((q_n, STATE_W), jnp.float32)],
    )(-2.0 * desc_nir, ka)
    return out[0, 0]
